# Initial kernel scaffold; baseline (speedup 1.0000x reference)
import jax
import jax.numpy as jnp
from jax.experimental import pallas as pl


def kernel(features, edge_row, edge_col, edge_val, W1, b1, W2, b2):
    raise NotImplementedError("write your pallas kernel here")


# speedup vs baseline: 4.7860x; 4.7860x over previous
import functools

import jax
import jax.numpy as jnp
from jax import lax
from jax.experimental import pallas as pl
from jax.experimental.pallas import tpu as pltpu
from jax.experimental.pallas import tpu_sc as plsc

NC = 2
NS = 16
L = 16
K = 128


def _make_sc_spmm(n, half, ch):
    npt = n // NS
    n_chunks = npt // K
    mesh = plsc.VectorSubcoreMesh(
        core_axis_name="c", subcore_axis_name="s", num_cores=NC,
        num_subcores=NS)

    @functools.partial(
        pl.kernel,
        out_type=[
            jax.ShapeDtypeStruct((NC, n, half), jnp.float32),
            jax.ShapeDtypeStruct((NC, n, half), jnp.float32),
        ],
        mesh=mesh,
        scratch_types=[
            pltpu.VMEM((ch, K), jnp.int32),
            pltpu.VMEM((ch, K), jnp.int32),
            pltpu.VMEM((ch, K), jnp.float32),
            pltpu.VMEM((K, half), jnp.float32),
            pltpu.VMEM((K, half), jnp.float32),
            pltpu.VMEM_SHARED((n, half), jnp.float32),
            pltpu.SemaphoreType.DMA,
        ],
        compiler_params=pltpu.CompilerParams(use_tc_tiling_on_sc=False),
    )
    def sc_kernel(table_h, col_h, row_h, val_h, o1_h, o2_h,
                  col_v, row_v, val_v, rows_v, m_v, acc, sem):
        cid = lax.axis_index("c")
        sid = lax.axis_index("s")

        pltpu.sync_copy(col_h.at[cid, sid], col_v)
        pltpu.sync_copy(row_h.at[sid], row_v)
        pltpu.sync_copy(val_h.at[sid], val_v)

        base = sid * npt
        zero = jnp.zeros((L,), jnp.float32)

        def zero_acc():
            def zb(k, carry):
                for j in range(half // L):
                    m_v[k, pl.ds(j * L, L)] = zero
                return carry

            lax.fori_loop(0, K, zb, 0)
            for i in range(n_chunks):
                pltpu.sync_copy(m_v, acc.at[pl.ds(base + i * K, K)])

        def spmm_pass(square, o_h):
            zero_acc()
            plsc.subcore_barrier()

            def chunk_body(c, carry):
                pltpu.async_copy(table_h.at[col_v.at[c]], rows_v, sem).wait()

                def group(g, carry2):
                    vv = val_v[c, pl.ds(g * L, L)]
                    for k in range(L):
                        v = vv[k]
                        kk = g * L + k
                        for j in range(half // L):
                            r = rows_v[kk, pl.ds(j * L, L)]
                            m = r * v
                            if square:
                                m = m * r
                            m_v[kk, pl.ds(j * L, L)] = m
                    return carry2

                lax.fori_loop(0, K // L, group, 0)
                pltpu.sync_copy(m_v, acc.at[row_v.at[c]], add=True)
                return carry

            lax.fori_loop(0, ch, chunk_body, 0)
            plsc.subcore_barrier()
            for i in range(n_chunks):
                pltpu.sync_copy(acc.at[pl.ds(base + i * K, K)],
                                o_h.at[cid, pl.ds(base + i * K, K)])
            plsc.subcore_barrier()

        spmm_pass(False, o1_h)
        spmm_pass(True, o2_h)

    return sc_kernel


def _tc_tail(o1, o2, f, w1t, w2t, b1, b2, n, d, half):
    blk = 400
    grid = (n // blk,)

    def body(o1a, o1b, o2a, o2b, fr, w1, w2, bb1, bb2, out):
        agg1 = jnp.concatenate([o1a[...], o1b[...]], axis=1) + fr[...]
        x1 = jnp.dot(agg1, w1[...], preferred_element_type=jnp.float32) + bb1[...]
        agg2 = jnp.concatenate([o2a[...], o2b[...]], axis=1)
        x2 = jnp.dot(agg2, w2[...], preferred_element_type=jnp.float32) + bb2[...]
        y1 = jnp.where(x1 > 0, x1, 0.01 * x1)
        y2 = jnp.where(x2 > 0, x2, 0.01 * x2)
        out[...] = y1 + y2

    hs = pl.BlockSpec((blk, half), lambda i: (i, 0))
    fs = pl.BlockSpec((blk, d), lambda i: (i, 0))
    ws = pl.BlockSpec((d, d), lambda i: (0, 0))
    bs = pl.BlockSpec((1, d), lambda i: (0, 0))
    return pl.pallas_call(
        body,
        grid=grid,
        in_specs=[hs, hs, hs, hs, fs, ws, ws, bs, bs],
        out_specs=fs,
        out_shape=jax.ShapeDtypeStruct((n, d), jnp.float32),
    )(o1[0], o1[1], o2[0], o2[1], f, w1t, w2t, b1, b2)


def kernel(features, edge_row, edge_col, edge_val, W1, b1, W2, b2):
    n, d = features.shape
    e = edge_row.shape[0]
    half = d // 2

    gran = NS * K
    e_pad = -(-e // gran) * gran
    pad = e_pad - e
    ch = e_pad // (NS * K)
    col_p = jnp.pad(edge_col, (0, pad))
    row_p = jnp.pad(edge_row, (0, pad))
    val_p = jnp.pad(edge_val, (0, pad))
    col2 = jnp.stack([col_p, col_p + n]).reshape(NC, NS, ch, K)
    row_r = row_p.reshape(NS, ch, K)
    val_r = val_p.reshape(NS, ch, K)
    table = features.reshape(n, NC, half).transpose(1, 0, 2).reshape(NC * n, half)

    n_acc = -(-n // (NS * K)) * NS * K

    o1, o2 = _make_sc_spmm(n_acc, half, ch)(table, col2, row_r, val_r)

    return _tc_tail(o1, o2, features, W1.T, W2.T,
                    b1.reshape(1, d), b2.reshape(1, d), n, d, half)
